# baseline (speedup 1.0000x reference)
import math

import jax
import jax.numpy as jnp
from jax import lax
from jax.experimental import pallas as pl
from jax.experimental.pallas import tpu as pltpu

_NUM_CLASSES = 100000
_BATCH = 1024
_SCALE = 64.0
_MARGIN = 0.5
_COS_M = math.cos(_MARGIN)
_SIN_M = math.sin(_MARGIN)
_THRESHOLD = math.cos(math.pi - _MARGIN)
_MM = math.sin(math.pi - _MARGIN) * _MARGIN

_CB = 2048
_NJ = (_NUM_CLASSES + _CB - 1) // _CB


def _f16_roundtrip(x):
    bits = lax.bitcast_convert_type(x, jnp.int32)
    rb = (bits + 0xFFF + ((bits >> 13) & 1)) & ~0x1FFF
    normal = lax.bitcast_convert_type(rb, jnp.float32)
    half = jnp.float32(0.5)
    mag = jnp.abs(x)
    magq = (mag + half) - half
    sub = jnp.where(x < 0, -magq, magq)
    return jnp.where(mag < jnp.float32(2.0 ** -14), sub, normal)


def _prologue_body(lbl_ref, lbl2d_ref, ct_hbm, t_ref, ctm_ref, ftl_ref, th_ref,
                   tl_v, sem):
    def issue(i, carry):
        c128 = pl.multiple_of((lbl_ref[i] // 128) * 128, 128)
        r8 = pl.multiple_of((i // 8) * 8, 8)
        l0 = pl.multiple_of((i % 8) * 128, 128)
        pltpu.make_async_copy(
            ct_hbm.at[pl.ds(r8, 8), pl.ds(c128, 128)],
            tl_v.at[pl.ds(r8, 8), pl.ds(l0, 128)],
            sem,
        ).start()
        return carry

    lax.fori_loop(0, _BATCH, issue, 0, unroll=32)

    pltpu.make_async_copy(
        ct_hbm.at[:, pl.ds(0, 1024)], tl_v, sem
    ).wait()

    lbl2d = lbl2d_ref[...]
    rowrem = lax.broadcasted_iota(jnp.int32, (_BATCH, 1), 0) & 7
    pos = rowrem * 128 + (lbl2d & 127)
    sel = lax.broadcasted_iota(jnp.int32, (_BATCH, 1024), 1) == pos
    tl = jnp.sum(jnp.where(sel, tl_v[...], 0.0), axis=1, keepdims=True)
    tl = jnp.clip(tl, -1.0, 1.0)
    t_new = jnp.mean(tl) * 0.001 + (1.0 - 0.001) * t_ref[0]
    t_new11 = jnp.full((1, 1), t_new, jnp.float32)
    sin_t = jnp.sqrt(1.0 - tl * tl)
    ctm = tl * _COS_M - sin_t * _SIN_M
    ftl = jnp.where(tl > _THRESHOLD, ctm, tl - _MM)
    ctm_ref[...] = ctm
    ftl_ref[...] = _f16_roundtrip(ftl) * _SCALE
    th_ref[...] = _f16_roundtrip(t_new11)


def _prologue(label, label2d, cos_theta, t1):
    return pl.pallas_call(
        _prologue_body,
        in_specs=[
            pl.BlockSpec(memory_space=pltpu.SMEM),
            pl.BlockSpec((_BATCH, 1), lambda: (0, 0)),
            pl.BlockSpec(memory_space=pl.ANY),
            pl.BlockSpec(memory_space=pltpu.SMEM),
        ],
        out_specs=[
            pl.BlockSpec((_BATCH, 1), lambda: (0, 0)),
            pl.BlockSpec((_BATCH, 1), lambda: (0, 0)),
            pl.BlockSpec((1, 1), lambda: (0, 0)),
        ],
        out_shape=[
            jax.ShapeDtypeStruct((_BATCH, 1), jnp.float32),
            jax.ShapeDtypeStruct((_BATCH, 1), jnp.float32),
            jax.ShapeDtypeStruct((1, 1), jnp.float32),
        ],
        scratch_shapes=[
            pltpu.VMEM((_BATCH, 1024), jnp.float32),
            pltpu.SemaphoreType.DMA,
        ],
    )(label, label2d, cos_theta, t1)


def _stream_body(ct_ref, ctm_ref, ftl_ref, lbl_ref, th_ref, out_ref):
    j = pl.program_id(0)
    ct = jnp.clip(ct_ref[...], -1.0, 1.0)
    val = jnp.where(ct > ctm_ref[...], ct * (th_ref[...] + ct), ct) * _SCALE
    col = j * _CB + lax.broadcasted_iota(jnp.int32, ct.shape, 1)
    out_ref[...] = jnp.where(col == lbl_ref[...], ftl_ref[...], val)


def _stream(cos_theta, ctm, ftl, label2d, th):
    return pl.pallas_call(
        _stream_body,
        grid=(_NJ,),
        in_specs=[
            pl.BlockSpec((_BATCH, _CB), lambda j: (0, j)),
            pl.BlockSpec((_BATCH, 1), lambda j: (0, 0)),
            pl.BlockSpec((_BATCH, 1), lambda j: (0, 0)),
            pl.BlockSpec((_BATCH, 1), lambda j: (0, 0)),
            pl.BlockSpec((1, 1), lambda j: (0, 0)),
        ],
        out_specs=pl.BlockSpec((_BATCH, _CB), lambda j: (0, j)),
        out_shape=jax.ShapeDtypeStruct((_BATCH, _NUM_CLASSES), jnp.float32),
    )(cos_theta, ctm, ftl, label2d, th)


def kernel(cos_theta, label, t):
    label2d = label.reshape(_BATCH, 1)
    ctm, ftl, th = _prologue(label, label2d, cos_theta, t.reshape(1))
    return _stream(cos_theta, ctm, ftl, label2d, th)


# speedup vs baseline: 1.0048x; 1.0048x over previous
import math

import jax
import jax.numpy as jnp
from jax import lax
from jax.experimental import pallas as pl
from jax.experimental.pallas import tpu as pltpu

_NUM_CLASSES = 100000
_BATCH = 1024
_SCALE = 64.0
_MARGIN = 0.5
_COS_M = math.cos(_MARGIN)
_SIN_M = math.sin(_MARGIN)
_THRESHOLD = math.cos(math.pi - _MARGIN)
_MM = math.sin(math.pi - _MARGIN) * _MARGIN

_CB = 3072
_NJ = (_NUM_CLASSES + _CB - 1) // _CB


def _f16_roundtrip(x):
    bits = lax.bitcast_convert_type(x, jnp.int32)
    rb = (bits + 0xFFF + ((bits >> 13) & 1)) & ~0x1FFF
    normal = lax.bitcast_convert_type(rb, jnp.float32)
    half = jnp.float32(0.5)
    mag = jnp.abs(x)
    magq = (mag + half) - half
    sub = jnp.where(x < 0, -magq, magq)
    return jnp.where(mag < jnp.float32(2.0 ** -14), sub, normal)


def _prologue_body(lbl_ref, lbl2d_ref, ct_hbm, t_ref, ctm_ref, ftl_ref, th_ref,
                   tl_v, sem):
    def issue(i, carry):
        c128 = pl.multiple_of((lbl_ref[i] // 128) * 128, 128)
        r8 = pl.multiple_of((i // 8) * 8, 8)
        l0 = pl.multiple_of((i % 8) * 128, 128)
        pltpu.make_async_copy(
            ct_hbm.at[pl.ds(r8, 8), pl.ds(c128, 128)],
            tl_v.at[pl.ds(r8, 8), pl.ds(l0, 128)],
            sem,
        ).start()
        return carry

    lax.fori_loop(0, _BATCH, issue, 0, unroll=32)

    pltpu.make_async_copy(
        ct_hbm.at[:, pl.ds(0, 1024)], tl_v, sem
    ).wait()

    lbl2d = lbl2d_ref[...]
    rowrem = lax.broadcasted_iota(jnp.int32, (_BATCH, 1), 0) & 7
    pos = rowrem * 128 + (lbl2d & 127)
    sel = lax.broadcasted_iota(jnp.int32, (_BATCH, 1024), 1) == pos
    tl = jnp.sum(jnp.where(sel, tl_v[...], 0.0), axis=1, keepdims=True)
    tl = jnp.clip(tl, -1.0, 1.0)
    t_new = jnp.mean(tl) * 0.001 + (1.0 - 0.001) * t_ref[0]
    t_new11 = jnp.full((1, 1), t_new, jnp.float32)
    sin_t = jnp.sqrt(1.0 - tl * tl)
    ctm = tl * _COS_M - sin_t * _SIN_M
    ftl = jnp.where(tl > _THRESHOLD, ctm, tl - _MM)
    ctm_ref[...] = ctm
    ftl_ref[...] = _f16_roundtrip(ftl) * _SCALE
    th_ref[...] = _f16_roundtrip(t_new11)


def _prologue(label, label2d, cos_theta, t1):
    return pl.pallas_call(
        _prologue_body,
        in_specs=[
            pl.BlockSpec(memory_space=pltpu.SMEM),
            pl.BlockSpec((_BATCH, 1), lambda: (0, 0)),
            pl.BlockSpec(memory_space=pl.ANY),
            pl.BlockSpec(memory_space=pltpu.SMEM),
        ],
        out_specs=[
            pl.BlockSpec((_BATCH, 1), lambda: (0, 0)),
            pl.BlockSpec((_BATCH, 1), lambda: (0, 0)),
            pl.BlockSpec((1, 1), lambda: (0, 0)),
        ],
        out_shape=[
            jax.ShapeDtypeStruct((_BATCH, 1), jnp.float32),
            jax.ShapeDtypeStruct((_BATCH, 1), jnp.float32),
            jax.ShapeDtypeStruct((1, 1), jnp.float32),
        ],
        scratch_shapes=[
            pltpu.VMEM((_BATCH, 1024), jnp.float32),
            pltpu.SemaphoreType.DMA,
        ],
    )(label, label2d, cos_theta, t1)


def _stream_body(ct_ref, ctm_ref, ftl_ref, lbl_ref, th_ref, out_ref):
    j = pl.program_id(0)
    ct = jnp.clip(ct_ref[...], -1.0, 1.0)
    val = jnp.where(ct > ctm_ref[...], ct * (th_ref[...] + ct), ct) * _SCALE
    col = j * _CB + lax.broadcasted_iota(jnp.int32, ct.shape, 1)
    out_ref[...] = jnp.where(col == lbl_ref[...], ftl_ref[...], val)


def _stream(cos_theta, ctm, ftl, label2d, th):
    return pl.pallas_call(
        _stream_body,
        grid=(_NJ,),
        in_specs=[
            pl.BlockSpec((_BATCH, _CB), lambda j: (0, j)),
            pl.BlockSpec((_BATCH, 1), lambda j: (0, 0)),
            pl.BlockSpec((_BATCH, 1), lambda j: (0, 0)),
            pl.BlockSpec((_BATCH, 1), lambda j: (0, 0)),
            pl.BlockSpec((1, 1), lambda j: (0, 0)),
        ],
        out_specs=pl.BlockSpec((_BATCH, _CB), lambda j: (0, j)),
        out_shape=jax.ShapeDtypeStruct((_BATCH, _NUM_CLASSES), jnp.float32),
    )(cos_theta, ctm, ftl, label2d, th)


def kernel(cos_theta, label, t):
    label2d = label.reshape(_BATCH, 1)
    ctm, ftl, th = _prologue(label, label2d, cos_theta, t.reshape(1))
    return _stream(cos_theta, ctm, ftl, label2d, th)
